# baseline (speedup 1.0000x reference)
import functools

import jax
import jax.numpy as jnp
from jax import lax
from jax.experimental import pallas as pl
from jax.experimental.pallas import tpu as pltpu
from jax.experimental.pallas import tpu_sc as plsc

N_NODES = 100000
IN_FEATS = 512
NUM_SEGMENTS = 1024

_NC = 2
_NS = 16
_NW = _NC * _NS
_PER_TILE = 3128
_LAST_COUNT = N_NODES - (_NW - 1) * _PER_TILE
_BUF = _PER_TILE + 24
_NVEC = (_PER_TILE + 15) // 16
_SENTINEL = 1 << 30

_R = 2000
_NB = N_NODES // _R
_W = 64
_EPADN = NUM_SEGMENTS + 1 + _W + 8
_BPAD = NUM_SEGMENTS + _W


def _seg_ends_body(batch_hbm, out_hbm, buf, eloc, stage, merged, shared):
    cid = lax.axis_index("c")
    sid = lax.axis_index("s")
    wid = sid * _NC + cid
    start = wid * _PER_TILE
    count = jnp.where(wid == _NW - 1, _LAST_COUNT, _PER_TILE)

    @pl.when(wid < _NW - 1)
    def _():
        pltpu.sync_copy(batch_hbm.at[pl.ds(start, _PER_TILE + 8)],
                        buf.at[pl.ds(0, _PER_TILE + 8)])

    @pl.when(wid == _NW - 1)
    def _():
        pltpu.sync_copy(batch_hbm.at[pl.ds(start, _LAST_COUNT)],
                        buf.at[pl.ds(0, _LAST_COUNT)])
        base = (_LAST_COUNT // 16) * 16
        v = buf[pl.ds(base, 16)]
        pos = lax.iota(jnp.int32, 16) + base
        buf[pl.ds(base, 16)] = jnp.where(pos == _LAST_COUNT, _SENTINEL, v)

    def _zero(k, _):
        eloc[pl.ds(k * 16, 16)] = jnp.zeros((16,), jnp.int32)
        return 0
    lax.fori_loop(0, NUM_SEGMENTS // 16, _zero, 0)

    def _scan(j, _):
        a = buf[pl.ds(j * 16, 16)]
        b = buf[pl.ds(j * 16 + 1, 16)]
        lane = lax.iota(jnp.int32, 16)
        valid = (lane + j * 16) < count
        m = jnp.logical_and(a != b, valid)
        val = start + j * 16 + lane + 1
        plsc.store_scatter(eloc, [a], val, mask=m)
        return 0
    lax.fori_loop(0, _NVEC, _scan, 0)

    pltpu.sync_copy(eloc, shared.at[sid])
    plsc.subcore_barrier()

    @pl.when(sid == 0)
    def _merge():
        pltpu.sync_copy(shared, stage)

        def _mx(k, _):
            v = stage[0, pl.ds(k * 16, 16)]
            for t in range(1, _NS):
                v = jnp.maximum(v, stage[t, pl.ds(k * 16, 16)])
            merged[pl.ds(k * 16, 16)] = v
            return 0
        lax.fori_loop(0, NUM_SEGMENTS // 16, _mx, 0)

        def _cm(k, c):
            v = jnp.maximum(merged[pl.ds(k * 16, 16)], c)
            s = plsc.cummax(v)
            merged[pl.ds(k * 16, 16)] = s
            last = lax.reduce_max(s, axes=(0,))
            return jnp.broadcast_to(last, (16,))
        lax.fori_loop(0, NUM_SEGMENTS // 16, _cm,
                      jnp.zeros((16,), jnp.int32), unroll=False)

        pltpu.sync_copy(merged, out_hbm.at[cid])


def _seg_ends(batch):
    kfn = functools.partial(
        pl.kernel,
        out_type=jax.ShapeDtypeStruct((_NC, NUM_SEGMENTS), jnp.int32),
        mesh=plsc.VectorSubcoreMesh(core_axis_name="c", subcore_axis_name="s"),
        compiler_params=pltpu.CompilerParams(needs_layout_passes=False),
        scratch_types=[
            pltpu.VMEM((_BUF,), jnp.int32),
            pltpu.VMEM((NUM_SEGMENTS,), jnp.int32),
            pltpu.VMEM((_NS, NUM_SEGMENTS), jnp.int32),
            pltpu.VMEM((NUM_SEGMENTS,), jnp.int32),
            pltpu.VMEM_SHARED((_NS, NUM_SEGMENTS), jnp.int32),
        ],
    )(_seg_ends_body)
    return kfn(batch)


def _main_body(eps_ref, evm_ref, x_ref, w_ref, b_ref, out_ref, bbuf):
    i = pl.program_id(0)

    @pl.when(i == 0)
    def _init():
        bbuf[...] = jnp.zeros_like(bbuf)

    xb = x_ref[...]
    z = lax.dot_general(w_ref[...], xb, (((0,), (1,)), ((), ())),
                        preferred_element_type=jnp.float32)
    wgt = jax.nn.sigmoid(z + b_ref[0, 0])

    blk_start = i * _R
    blk_end = (i + 1) * _R

    def _eps(u):
        return jnp.maximum(eps_ref[0, u], eps_ref[1, u])

    def _bs(thresh, strict):
        def body(_, lohi):
            lo, hi = lohi
            mid = (lo + hi) // 2
            v = _eps(mid)
            pred = jnp.where(strict, v <= thresh, v < thresh)
            return jnp.where(pred, mid + 1, lo), jnp.where(pred, hi, mid)
        lo, _ = lax.fori_loop(0, 11, body,
                              (jnp.int32(0), jnp.int32(NUM_SEGMENTS + 1)))
        return lo

    seg_a = ((_bs(blk_start, True) - 1) // 8) * 8
    seg_c = _bs(blk_end, False)
    ntrips = (seg_c - seg_a + _W - 1) // _W

    jglob = lax.broadcasted_iota(jnp.int32, (_W, _R), 1) + blk_start

    def _window(t, _):
        s0 = pl.multiple_of(seg_a + t * _W, 8)
        evc = jnp.maximum(evm_ref[0, pl.ds(s0, _W + 8), :],
                          evm_ref[1, pl.ds(s0, _W + 8), :])
        lo0 = evc[0:_W, :]
        hi0 = evc[1:_W + 1, :]
        ind = jnp.logical_and(jglob >= lo0, jglob < hi0)
        oh = jnp.where(ind, wgt, 0.0)
        bbuf[pl.ds(s0, _W), :] += jnp.dot(oh, xb,
                                          preferred_element_type=jnp.float32)
        return 0
    lax.fori_loop(0, ntrips, _window, 0)

    @pl.when(i == _NB - 1)
    def _final():
        out_ref[...] = bbuf[0:NUM_SEGMENTS, :]


def _main(x, W, b, e2):
    zero = jnp.zeros((_NC, 1), jnp.int32)
    eps = jnp.concatenate([zero, e2], axis=1)
    evm = jnp.pad(eps, ((0, 0), (0, _EPADN - (NUM_SEGMENTS + 1))),
                  constant_values=N_NODES)
    evm = evm.reshape(_NC, _EPADN, 1)
    return pl.pallas_call(
        _main_body,
        grid=(_NB,),
        in_specs=[
            pl.BlockSpec(memory_space=pltpu.SMEM),
            pl.BlockSpec((_NC, _EPADN, 1), lambda i: (0, 0, 0)),
            pl.BlockSpec((_R, IN_FEATS), lambda i: (i, 0)),
            pl.BlockSpec((IN_FEATS, 1), lambda i: (0, 0)),
            pl.BlockSpec(memory_space=pltpu.SMEM),
        ],
        out_specs=pl.BlockSpec((NUM_SEGMENTS, IN_FEATS), lambda i: (0, 0)),
        out_shape=jax.ShapeDtypeStruct((NUM_SEGMENTS, IN_FEATS), jnp.float32),
        scratch_shapes=[
            pltpu.VMEM((_BPAD, IN_FEATS), jnp.float32),
        ],
    )(eps, evm, x, W, b.reshape(1, 1))


def kernel(x, batch, W, b):
    e2 = _seg_ends(batch)
    return _main(x, W, b, e2)


# speedup vs baseline: 1.1603x; 1.1603x over previous
import functools

import jax
import jax.numpy as jnp
from jax import lax
from jax.experimental import pallas as pl
from jax.experimental.pallas import tpu as pltpu
from jax.experimental.pallas import tpu_sc as plsc

N_NODES = 100000
IN_FEATS = 512
NUM_SEGMENTS = 1024

_NC = 2
_NS = 16
_NW = _NC * _NS
_PER_TILE = 3128
_LAST_COUNT = N_NODES - (_NW - 1) * _PER_TILE
_BUF = _PER_TILE + 24
_NVEC = (_PER_TILE + 15) // 16
_SENTINEL = 1 << 30

_R = 10000
_NB = N_NODES // _R
_W = 64
_EPADN = NUM_SEGMENTS + 1 + _W + 8
_BPAD = NUM_SEGMENTS + _W


def _seg_ends_body(batch_hbm, out_hbm, buf, eloc, stage, merged, shared):
    cid = lax.axis_index("c")
    sid = lax.axis_index("s")
    wid = sid * _NC + cid
    start = wid * _PER_TILE
    count = jnp.where(wid == _NW - 1, _LAST_COUNT, _PER_TILE)

    @pl.when(wid < _NW - 1)
    def _():
        pltpu.sync_copy(batch_hbm.at[pl.ds(start, _PER_TILE + 8)],
                        buf.at[pl.ds(0, _PER_TILE + 8)])

    @pl.when(wid == _NW - 1)
    def _():
        pltpu.sync_copy(batch_hbm.at[pl.ds(start, _LAST_COUNT)],
                        buf.at[pl.ds(0, _LAST_COUNT)])
        base = (_LAST_COUNT // 16) * 16
        v = buf[pl.ds(base, 16)]
        pos = lax.iota(jnp.int32, 16) + base
        buf[pl.ds(base, 16)] = jnp.where(pos == _LAST_COUNT, _SENTINEL, v)

    def _zero(k, _):
        eloc[pl.ds(k * 16, 16)] = jnp.zeros((16,), jnp.int32)
        return 0
    lax.fori_loop(0, NUM_SEGMENTS // 16, _zero, 0)

    def _scan(j, _):
        a = buf[pl.ds(j * 16, 16)]
        b = buf[pl.ds(j * 16 + 1, 16)]
        lane = lax.iota(jnp.int32, 16)
        valid = (lane + j * 16) < count
        m = jnp.logical_and(a != b, valid)
        val = start + j * 16 + lane + 1
        plsc.store_scatter(eloc, [a], val, mask=m)
        return 0
    lax.fori_loop(0, _NVEC, _scan, 0)

    pltpu.sync_copy(eloc, shared.at[sid])
    plsc.subcore_barrier()

    @pl.when(sid == 0)
    def _merge():
        pltpu.sync_copy(shared, stage)

        def _mx(k, _):
            v = stage[0, pl.ds(k * 16, 16)]
            for t in range(1, _NS):
                v = jnp.maximum(v, stage[t, pl.ds(k * 16, 16)])
            merged[pl.ds(k * 16, 16)] = v
            return 0
        lax.fori_loop(0, NUM_SEGMENTS // 16, _mx, 0)

        def _cm(k, c):
            v = jnp.maximum(merged[pl.ds(k * 16, 16)], c)
            s = plsc.cummax(v)
            merged[pl.ds(k * 16, 16)] = s
            last = lax.reduce_max(s, axes=(0,))
            return jnp.broadcast_to(last, (16,))
        lax.fori_loop(0, NUM_SEGMENTS // 16, _cm,
                      jnp.zeros((16,), jnp.int32), unroll=False)

        pltpu.sync_copy(merged, out_hbm.at[cid])


def _seg_ends(batch):
    kfn = functools.partial(
        pl.kernel,
        out_type=jax.ShapeDtypeStruct((_NC, NUM_SEGMENTS), jnp.int32),
        mesh=plsc.VectorSubcoreMesh(core_axis_name="c", subcore_axis_name="s"),
        compiler_params=pltpu.CompilerParams(needs_layout_passes=False),
        scratch_types=[
            pltpu.VMEM((_BUF,), jnp.int32),
            pltpu.VMEM((NUM_SEGMENTS,), jnp.int32),
            pltpu.VMEM((_NS, NUM_SEGMENTS), jnp.int32),
            pltpu.VMEM((NUM_SEGMENTS,), jnp.int32),
            pltpu.VMEM_SHARED((_NS, NUM_SEGMENTS), jnp.int32),
        ],
    )(_seg_ends_body)
    return kfn(batch)


def _main_body(eps_ref, evm_ref, x_ref, w_ref, b_ref, out_ref, bbuf):
    i = pl.program_id(0)

    @pl.when(i == 0)
    def _init():
        bbuf[...] = jnp.zeros_like(bbuf)

    xb = x_ref[...]
    z = lax.dot_general(w_ref[...], xb, (((0,), (1,)), ((), ())),
                        preferred_element_type=jnp.float32)
    wgt = jax.nn.sigmoid(z + b_ref[0, 0])

    blk_start = i * _R
    blk_end = (i + 1) * _R

    def _eps(u):
        return jnp.maximum(eps_ref[0, u], eps_ref[1, u])

    def _bs(thresh, strict):
        def body(_, lohi):
            lo, hi = lohi
            mid = (lo + hi) // 2
            v = _eps(mid)
            pred = jnp.where(strict, v <= thresh, v < thresh)
            return jnp.where(pred, mid + 1, lo), jnp.where(pred, hi, mid)
        lo, _ = lax.fori_loop(0, 11, body,
                              (jnp.int32(0), jnp.int32(NUM_SEGMENTS + 1)))
        return lo

    seg_a = ((_bs(blk_start, True) - 1) // 8) * 8
    seg_c = _bs(blk_end, False)
    ntrips = (seg_c - seg_a + _W - 1) // _W

    jglob = lax.broadcasted_iota(jnp.int32, (_W, _R), 1) + blk_start

    def _window(t, _):
        s0 = pl.multiple_of(seg_a + t * _W, 8)
        evc = jnp.maximum(evm_ref[0, pl.ds(s0, _W + 8), :],
                          evm_ref[1, pl.ds(s0, _W + 8), :])
        lo0 = evc[0:_W, :]
        hi0 = evc[1:_W + 1, :]
        ind = jnp.logical_and(jglob >= lo0, jglob < hi0)
        oh = jnp.where(ind, wgt, 0.0)
        bbuf[pl.ds(s0, _W), :] += jnp.dot(oh, xb,
                                          preferred_element_type=jnp.float32)
        return 0
    lax.fori_loop(0, ntrips, _window, 0)

    @pl.when(i == _NB - 1)
    def _final():
        out_ref[...] = bbuf[0:NUM_SEGMENTS, :]


def _main(x, W, b, e2):
    zero = jnp.zeros((_NC, 1), jnp.int32)
    eps = jnp.concatenate([zero, e2], axis=1)
    evm = jnp.pad(eps, ((0, 0), (0, _EPADN - (NUM_SEGMENTS + 1))),
                  constant_values=N_NODES)
    evm = evm.reshape(_NC, _EPADN, 1)
    return pl.pallas_call(
        _main_body,
        grid=(_NB,),
        in_specs=[
            pl.BlockSpec(memory_space=pltpu.SMEM),
            pl.BlockSpec((_NC, _EPADN, 1), lambda i: (0, 0, 0)),
            pl.BlockSpec((_R, IN_FEATS), lambda i: (i, 0)),
            pl.BlockSpec((IN_FEATS, 1), lambda i: (0, 0)),
            pl.BlockSpec(memory_space=pltpu.SMEM),
        ],
        out_specs=pl.BlockSpec((NUM_SEGMENTS, IN_FEATS), lambda i: (0, 0)),
        out_shape=jax.ShapeDtypeStruct((NUM_SEGMENTS, IN_FEATS), jnp.float32),
        scratch_shapes=[
            pltpu.VMEM((_BPAD, IN_FEATS), jnp.float32),
        ],
    )(eps, evm, x, W, b.reshape(1, 1))


def kernel(x, batch, W, b):
    e2 = _seg_ends(batch)
    return _main(x, W, b, e2)


# speedup vs baseline: 1.2332x; 1.0629x over previous
import functools

import jax
import jax.numpy as jnp
from jax import lax
from jax.experimental import pallas as pl
from jax.experimental.pallas import tpu as pltpu
from jax.experimental.pallas import tpu_sc as plsc

N_NODES = 100000
IN_FEATS = 512
NUM_SEGMENTS = 1024

_NC = 2
_NS = 16
_NW = _NC * _NS
_PER_TILE = 3128
_LAST_COUNT = N_NODES - (_NW - 1) * _PER_TILE
_BUF = _PER_TILE + 24
_NVEC = (_PER_TILE + 15) // 16
_SENTINEL = 1 << 30

_R = 5000
_NB = N_NODES // _R
_W = 64
_EPADN = NUM_SEGMENTS + 1 + _W + 8
_BPAD = NUM_SEGMENTS + _W


def _seg_ends_body(batch_hbm, out_hbm, buf, eloc, stage, merged, shared):
    cid = lax.axis_index("c")
    sid = lax.axis_index("s")
    wid = sid * _NC + cid
    start = wid * _PER_TILE
    count = jnp.where(wid == _NW - 1, _LAST_COUNT, _PER_TILE)

    @pl.when(wid < _NW - 1)
    def _():
        pltpu.sync_copy(batch_hbm.at[pl.ds(start, _PER_TILE + 8)],
                        buf.at[pl.ds(0, _PER_TILE + 8)])

    @pl.when(wid == _NW - 1)
    def _():
        pltpu.sync_copy(batch_hbm.at[pl.ds(start, _LAST_COUNT)],
                        buf.at[pl.ds(0, _LAST_COUNT)])
        base = (_LAST_COUNT // 16) * 16
        v = buf[pl.ds(base, 16)]
        pos = lax.iota(jnp.int32, 16) + base
        buf[pl.ds(base, 16)] = jnp.where(pos == _LAST_COUNT, _SENTINEL, v)

    def _zero(k, _):
        eloc[pl.ds(k * 16, 16)] = jnp.zeros((16,), jnp.int32)
        return 0
    lax.fori_loop(0, NUM_SEGMENTS // 16, _zero, 0)

    def _scan(j, _):
        a = buf[pl.ds(j * 16, 16)]
        b = buf[pl.ds(j * 16 + 1, 16)]
        lane = lax.iota(jnp.int32, 16)
        valid = (lane + j * 16) < count
        m = jnp.logical_and(a != b, valid)
        val = start + j * 16 + lane + 1
        plsc.store_scatter(eloc, [a], val, mask=m)
        return 0
    lax.fori_loop(0, _NVEC, _scan, 0)

    pltpu.sync_copy(eloc, shared.at[sid])
    plsc.subcore_barrier()

    @pl.when(sid == 0)
    def _merge():
        pltpu.sync_copy(shared, stage)

        def _mx(k, _):
            v = stage[0, pl.ds(k * 16, 16)]
            for t in range(1, _NS):
                v = jnp.maximum(v, stage[t, pl.ds(k * 16, 16)])
            merged[pl.ds(k * 16, 16)] = v
            return 0
        lax.fori_loop(0, NUM_SEGMENTS // 16, _mx, 0)

        def _cm(k, c):
            v = jnp.maximum(merged[pl.ds(k * 16, 16)], c)
            s = plsc.cummax(v)
            merged[pl.ds(k * 16, 16)] = s
            last = lax.reduce_max(s, axes=(0,))
            return jnp.broadcast_to(last, (16,))
        lax.fori_loop(0, NUM_SEGMENTS // 16, _cm,
                      jnp.zeros((16,), jnp.int32), unroll=False)

        pltpu.sync_copy(merged, out_hbm.at[cid])


def _seg_ends(batch):
    kfn = functools.partial(
        pl.kernel,
        out_type=jax.ShapeDtypeStruct((_NC, NUM_SEGMENTS), jnp.int32),
        mesh=plsc.VectorSubcoreMesh(core_axis_name="c", subcore_axis_name="s"),
        compiler_params=pltpu.CompilerParams(needs_layout_passes=False),
        scratch_types=[
            pltpu.VMEM((_BUF,), jnp.int32),
            pltpu.VMEM((NUM_SEGMENTS,), jnp.int32),
            pltpu.VMEM((_NS, NUM_SEGMENTS), jnp.int32),
            pltpu.VMEM((NUM_SEGMENTS,), jnp.int32),
            pltpu.VMEM_SHARED((_NS, NUM_SEGMENTS), jnp.int32),
        ],
    )(_seg_ends_body)
    return kfn(batch)


def _main_body(eps_ref, evm_ref, x_ref, w_ref, b_ref, out_ref, bbuf):
    i = pl.program_id(0)

    @pl.when(i == 0)
    def _init():
        bbuf[...] = jnp.zeros_like(bbuf)

    xb = x_ref[...]
    z = lax.dot_general(w_ref[...], xb, (((0,), (1,)), ((), ())),
                        preferred_element_type=jnp.float32)
    wgt = jax.nn.sigmoid(z + b_ref[0, 0])

    blk_start = i * _R
    blk_end = (i + 1) * _R

    def _eps(u):
        return jnp.maximum(eps_ref[0, u], eps_ref[1, u])

    def _bs(thresh, strict):
        def body(_, lohi):
            lo, hi = lohi
            mid = (lo + hi) // 2
            v = _eps(mid)
            pred = jnp.where(strict, v <= thresh, v < thresh)
            return jnp.where(pred, mid + 1, lo), jnp.where(pred, hi, mid)
        lo, _ = lax.fori_loop(0, 11, body,
                              (jnp.int32(0), jnp.int32(NUM_SEGMENTS + 1)))
        return lo

    seg_a = ((_bs(blk_start, True) - 1) // 8) * 8
    seg_c = _bs(blk_end, False)
    ntrips = (seg_c - seg_a + _W - 1) // _W

    jglob = lax.broadcasted_iota(jnp.int32, (_W, _R), 1) + blk_start

    def _window(t, _):
        s0 = pl.multiple_of(seg_a + t * _W, 8)
        evc = jnp.maximum(evm_ref[0, pl.ds(s0, _W + 8), :],
                          evm_ref[1, pl.ds(s0, _W + 8), :])
        lo0 = evc[0:_W, :]
        hi0 = evc[1:_W + 1, :]
        ind = jnp.logical_and(jglob >= lo0, jglob < hi0)
        oh = jnp.where(ind, wgt, 0.0)
        bbuf[pl.ds(s0, _W), :] += jnp.dot(oh, xb,
                                          preferred_element_type=jnp.float32)
        return 0
    lax.fori_loop(0, ntrips, _window, 0)

    @pl.when(i == _NB - 1)
    def _final():
        out_ref[...] = bbuf[0:NUM_SEGMENTS, :]


def _main(x, W, b, e2):
    zero = jnp.zeros((_NC, 1), jnp.int32)
    eps = jnp.concatenate([zero, e2], axis=1)
    evm = jnp.pad(eps, ((0, 0), (0, _EPADN - (NUM_SEGMENTS + 1))),
                  constant_values=N_NODES)
    evm = evm.reshape(_NC, _EPADN, 1)
    return pl.pallas_call(
        _main_body,
        grid=(_NB,),
        in_specs=[
            pl.BlockSpec(memory_space=pltpu.SMEM),
            pl.BlockSpec((_NC, _EPADN, 1), lambda i: (0, 0, 0)),
            pl.BlockSpec((_R, IN_FEATS), lambda i: (i, 0)),
            pl.BlockSpec((IN_FEATS, 1), lambda i: (0, 0)),
            pl.BlockSpec(memory_space=pltpu.SMEM),
        ],
        out_specs=pl.BlockSpec((NUM_SEGMENTS, IN_FEATS), lambda i: (0, 0)),
        out_shape=jax.ShapeDtypeStruct((NUM_SEGMENTS, IN_FEATS), jnp.float32),
        scratch_shapes=[
            pltpu.VMEM((_BPAD, IN_FEATS), jnp.float32),
        ],
    )(eps, evm, x, W, b.reshape(1, 1))


def kernel(x, batch, W, b):
    e2 = _seg_ends(batch)
    return _main(x, W, b, e2)
